# baseline (speedup 1.0000x reference)
import jax
import jax.numpy as jnp
from jax import lax
from jax.experimental import pallas as pl
from jax.experimental.pallas import tpu as pltpu
from jax.experimental.pallas import tpu_sc as plsc

H = 2048
E_DIM = 1024
N_SHARED = 2
N_ROUTED = 8
TOPK = 2
T = 4096
NP = T * TOPK

LANES = 128
TBLK = 256
VMAX = 48
NTILES = 32
TPT = T // NTILES
CH = 16


def _router_body(x_ref, w_ref, eidx_ref, prob_ref):
    logits = lax.dot_general(x_ref[...], w_ref[...], (((1,), (0,)), ((), ())),
                             preferred_element_type=jnp.float32)
    lane = lax.broadcasted_iota(jnp.int32, logits.shape, 1)
    valid = lane < N_ROUTED
    neg = jnp.full_like(logits, -jnp.inf)
    l = jnp.where(valid, logits, neg)
    m1 = jnp.max(l, axis=-1, keepdims=True)
    i1 = jnp.min(jnp.where(l == m1, lane, N_ROUTED + 7), axis=-1, keepdims=True)
    l2 = jnp.where(lane == i1, neg, l)
    m2 = jnp.max(l2, axis=-1, keepdims=True)
    i2 = jnp.min(jnp.where(l2 == m2, lane, N_ROUTED + 7), axis=-1, keepdims=True)
    z = jnp.sum(jnp.where(valid, jnp.exp(l - m1), 0.0), axis=-1, keepdims=True)
    p1 = 1.0 / z
    p2 = jnp.exp(m2 - m1) / z
    eidx_ref[...] = jnp.where(lane == 0, i1, jnp.where(lane == 1, i2, 0))
    prob_ref[...] = jnp.where(lane == 0, p1, jnp.where(lane == 1, p2, 0.0))


PPT2 = NP // NTILES


def _dispatch_body(ep_hbm, x_hbm, pos_hbm, vb_hbm, ve_hbm, vr0_hbm, vr1_hbm,
                   xg_hbm, ebuf, rankbuf, posbuf, cnt, vbv, vev, vr0v, vr1v,
                   xbuf, idxb, sem_in0, sem_in1, sem_sc0, sem_sc1):
    c = lax.axis_index("c")
    sid = lax.axis_index("s")
    wid = sid * 2 + c
    iota16 = lax.iota(jnp.int32, 16)
    zeros16 = jnp.zeros((16,), jnp.int32)

    if True:
        base = wid * PPT2
        pltpu.sync_copy(ep_hbm, ebuf)
        cnt[...] = zeros16
        for sub in range(PPT2 // 16):
            ev = ebuf[pl.ds(base + sub * 16, 16)]
            cntv = cnt[...]
            basec = zeros16
            cadd = zeros16
            for e in range(N_ROUTED):
                m = ev == e
                cs = plsc.cumsum(jnp.where(m, 1, 0))
                basec = jnp.where(m, cntv[e] + cs - 1, basec)
                ce = jnp.max(cs)
                cadd = cadd + jnp.where(iota16 == e, ce, 0)
            rankbuf[pl.ds(sub * 16, 16)] = basec
            cnt[...] = cntv + cadd
        def chunk(i, carry):
            te, pre = carry
            ev = ebuf[pl.ds(i * 16, 16)]
            cvec = zeros16
            for e in range(N_ROUTED):
                pc = plsc.all_reduce_population_count(ev == e)
                cvec = cvec + jnp.where(iota16 == e, pc, 0)
            te = te + cvec
            pre = pre + jnp.where(i * 16 < base, cvec, zeros16)
            return te, pre

        te, pre = lax.fori_loop(0, NP // 16, chunk, (zeros16, zeros16))
        excl = plsc.cumsum(te) - te
        myoffv = excl + pre

        @pl.when(wid == 0)
        def _visit_tables():
            o = [excl[e] for e in range(N_ROUTED)] + [jnp.int32(NP)]
            b0 = [o[e] // TBLK for e in range(N_ROUTED)]
            bend = [(o[e + 1] + TBLK - 1) // TBLK for e in range(N_ROUTED)]
            nb = [bend[e] - b0[e] for e in range(N_ROUTED)]
            V = [jnp.int32(0)]
            for e in range(N_ROUTED):
                V.append(V[-1] + nb[e])
            vtot = V[N_ROUTED]
            last_b = jnp.int32(0)
            last_e = jnp.int32(0)
            for e in range(N_ROUTED):
                nz = nb[e] > 0
                last_b = jnp.where(nz, bend[e] - 1, last_b)
                last_e = jnp.where(nz, e, last_e)
            for g in range(VMAX // 16):
                v = iota16 + 16 * g
                b = zeros16
                el = zeros16
                r0v = zeros16
                r1v = zeros16
                for e in range(N_ROUTED):
                    m = (v >= V[e]) & (v < V[e] + nb[e])
                    bb = b0[e] + (v - V[e])
                    rr0 = jnp.maximum(o[e] - bb * TBLK, 0)
                    rr1 = jnp.minimum(o[e + 1] - bb * TBLK, TBLK)
                    b = jnp.where(m, bb, b)
                    el = jnp.where(m, e, el)
                    r0v = jnp.where(m, rr0, r0v)
                    r1v = jnp.where(m, rr1, r1v)
                pad = v >= vtot
                b = jnp.where(pad, last_b, b)
                el = jnp.where(pad, last_e, el)
                r0v = jnp.where(pad, 0, r0v)
                r1v = jnp.where(pad, 0, r1v)
                sl = pl.ds(g * 16, 16)
                vbv[sl] = b
                vev[sl] = el
                vr0v[sl] = r0v
                vr1v[sl] = r1v
            pltpu.sync_copy(vbv, vb_hbm)
            pltpu.sync_copy(vev, ve_hbm)
            pltpu.sync_copy(vr0v, vr0_hbm)
            pltpu.sync_copy(vr1v, vr1_hbm)

        for sub in range(PPT2 // 16):
            ev = ebuf[pl.ds(base + sub * 16, 16)]
            basee = zeros16
            for e in range(N_ROUTED):
                basee = jnp.where(ev == e, myoffv[e], basee)
            posbuf[pl.ds(sub * 16, 16)] = basee + rankbuf[pl.ds(sub * 16, 16)]
        pltpu.sync_copy(posbuf, pos_hbm.at[pl.ds(base, PPT2)])

        tokb = base - jnp.where(base >= T, T, 0)
        nch = PPT2 // CH
        sem_in = (sem_in0, sem_in1)
        sem_sc = (sem_sc0, sem_sc1)

        def start_in(chv, s):
            pltpu.async_copy(x_hbm.at[pl.ds(tokb + chv * CH, CH)],
                             xbuf.at[s], sem_in[s])

        def wait_in(chv, s):
            pltpu.make_async_copy(x_hbm.at[pl.ds(tokb + chv * CH, CH)],
                                  xbuf.at[s], sem_in[s]).wait()

        def wait_sc(s):
            pltpu.make_async_copy(xbuf.at[s], xg_hbm.at[idxb.at[s]],
                                  sem_sc[s]).wait()

        start_in(0, 0)
        for chv in range(nch):
            s = chv % 2
            wait_in(chv, s)
            idxb.at[s][...] = posbuf[pl.ds(chv * CH, CH)]
            pltpu.async_copy(xbuf.at[s], xg_hbm.at[idxb.at[s]], sem_sc[s])
            if chv + 1 < nch:
                if chv >= 1:
                    wait_sc(1 - s)
                start_in(chv + 1, 1 - s)
        wait_sc((nch - 1) % 2)
        wait_sc(nch % 2)


def _dispatch(ep, x):
    mesh = plsc.VectorSubcoreMesh(core_axis_name="c", subcore_axis_name="s")
    i32 = jnp.int32
    f = pl.kernel(
        _dispatch_body,
        out_type=[
            jax.ShapeDtypeStruct((NP,), i32),
            jax.ShapeDtypeStruct((VMAX,), i32),
            jax.ShapeDtypeStruct((VMAX,), i32),
            jax.ShapeDtypeStruct((VMAX,), i32),
            jax.ShapeDtypeStruct((VMAX,), i32),
            jax.ShapeDtypeStruct((NP, H), jnp.float32),
        ],
        mesh=mesh,
        scratch_types=[
            pltpu.VMEM((NP,), i32),
            pltpu.VMEM((PPT2,), i32),
            pltpu.VMEM((PPT2,), i32),
            pltpu.VMEM((16,), i32),
            pltpu.VMEM((VMAX,), i32),
            pltpu.VMEM((VMAX,), i32),
            pltpu.VMEM((VMAX,), i32),
            pltpu.VMEM((VMAX,), i32),
            pltpu.VMEM((2, CH, H), jnp.float32),
            pltpu.VMEM((2, CH), i32),
            pltpu.SemaphoreType.DMA,
            pltpu.SemaphoreType.DMA,
            pltpu.SemaphoreType.DMA,
            pltpu.SemaphoreType.DMA,
        ],
        compiler_params=pltpu.CompilerParams(needs_layout_passes=False),
    )
    return f(ep, x)


def _gmm_body(vb_ref, ve_ref, vr0_ref, vr1_ref, xg_ref, up_ref, down_ref,
              ys_ref):
    v = pl.program_id(0)
    r0 = vr0_ref[v]
    r1 = vr1_ref[v]

    @pl.when(r1 > r0)
    def _():
        xb = xg_ref[...].astype(jnp.bfloat16)
        h = lax.dot_general(xb, up_ref[0], (((1,), (1,)), ((), ())),
                            preferred_element_type=jnp.float32)
        h = h * 0.5 * (1.0 + lax.erf(h * 0.7071067811865476))
        y = lax.dot_general(h.astype(jnp.bfloat16), down_ref[0],
                            (((1,), (1,)), ((), ())),
                            preferred_element_type=jnp.float32)
        rows = lax.broadcasted_iota(jnp.int32, (TBLK, H), 0)
        keep = (rows >= r0) & (rows < r1)
        ys_ref[...] = jnp.where(keep, y, ys_ref[...])


def _gmm(vb, ve, vr0, vr1, xg, up_bf, down_bf):
    grid_spec = pltpu.PrefetchScalarGridSpec(
        num_scalar_prefetch=4,
        grid=(VMAX,),
        in_specs=[
            pl.BlockSpec((TBLK, H), lambda v, vb, ve, r0, r1: (vb[v], 0)),
            pl.BlockSpec((1, E_DIM, H), lambda v, vb, ve, r0, r1: (ve[v], 0, 0)),
            pl.BlockSpec((1, H, E_DIM), lambda v, vb, ve, r0, r1: (ve[v], 0, 0)),
        ],
        out_specs=pl.BlockSpec((TBLK, H), lambda v, vb, ve, r0, r1: (vb[v], 0)),
    )
    return pl.pallas_call(
        _gmm_body,
        grid_spec=grid_spec,
        out_shape=jax.ShapeDtypeStruct((NP, H), jnp.float32),
    )(vb, ve, vr0, vr1, xg, up_bf, down_bf)


def _shared_body(x_ref, u_ref, d_ref, w_ref, out_ref, eidx_ref, prob_ref):
    _router_body(x_ref, w_ref, eidx_ref, prob_ref)
    xb = x_ref[...].astype(jnp.bfloat16)
    h = lax.dot_general(xb, u_ref[...], (((1,), (1,)), ((), ())),
                        preferred_element_type=jnp.float32)
    h = h * 0.5 * (1.0 + lax.erf(h * 0.7071067811865476))
    out_ref[...] = lax.dot_general(h.astype(jnp.bfloat16), d_ref[...],
                                   (((1,), (1,)), ((), ())),
                                   preferred_element_type=jnp.float32)


def _shared_router(x, u_bf, d_bf, router_w):
    su = N_SHARED * E_DIM
    w_pad = jnp.zeros((H, LANES), jnp.float32).at[:, :N_ROUTED].set(router_w.T)
    return pl.pallas_call(
        _shared_body,
        grid=(T // TBLK,),
        in_specs=[
            pl.BlockSpec((TBLK, H), lambda t: (t, 0)),
            pl.BlockSpec((su, H), lambda t: (0, 0)),
            pl.BlockSpec((H, su), lambda t: (0, 0)),
            pl.BlockSpec((H, LANES), lambda t: (0, 0)),
        ],
        out_specs=[
            pl.BlockSpec((TBLK, H), lambda t: (t, 0)),
            pl.BlockSpec((TBLK, LANES), lambda t: (t, 0)),
            pl.BlockSpec((TBLK, LANES), lambda t: (t, 0)),
        ],
        out_shape=[
            jax.ShapeDtypeStruct((T, H), jnp.float32),
            jax.ShapeDtypeStruct((T, LANES), jnp.int32),
            jax.ShapeDtypeStruct((T, LANES), jnp.float32),
        ],
    )(x, u_bf, d_bf, w_pad)


CH5 = 8


def _combine_body(sh_hbm, ys_hbm, pos_hbm, wp_hbm, out_hbm,
                  sbuf, g1, g2, posb1, posb2, wb1, wb2,
                  sem_in0, sem_in1, sem_out0, sem_out1):
    c = lax.axis_index("c")
    sid = lax.axis_index("s")
    wid = sid * 2 + c
    tbase = wid * TPT
    nch = TPT // CH5
    sem_in = (sem_in0, sem_in1)
    sem_out = (sem_out0, sem_out1)

    pltpu.sync_copy(pos_hbm.at[pl.ds(tbase, TPT)], posb1)
    pltpu.sync_copy(pos_hbm.at[pl.ds(T + tbase, TPT)], posb2)
    pltpu.sync_copy(wp_hbm.at[pl.ds(tbase, TPT)], wb1.at[pl.ds(0, TPT)])
    pltpu.sync_copy(wp_hbm.at[pl.ds(T + tbase, TPT)], wb2.at[pl.ds(0, TPT)])

    def start_in(chv, s):
        tb = tbase + chv * CH5
        pltpu.async_copy(ys_hbm.at[posb1.at[pl.ds(chv * CH5, CH5)]],
                         g1.at[s], sem_in[s])
        pltpu.async_copy(ys_hbm.at[posb2.at[pl.ds(chv * CH5, CH5)]],
                         g2.at[s], sem_in[s])
        pltpu.async_copy(sh_hbm.at[pl.ds(tb, CH5)], sbuf.at[s], sem_in[s])

    def wait_in(chv, s):
        tb = tbase + chv * CH5
        pltpu.make_async_copy(ys_hbm.at[posb1.at[pl.ds(chv * CH5, CH5)]],
                              g1.at[s], sem_in[s]).wait()
        pltpu.make_async_copy(ys_hbm.at[posb2.at[pl.ds(chv * CH5, CH5)]],
                              g2.at[s], sem_in[s]).wait()
        pltpu.make_async_copy(sh_hbm.at[pl.ds(tb, CH5)], sbuf.at[s],
                              sem_in[s]).wait()

    def wait_out(chv, s):
        tb = tbase + chv * CH5
        pltpu.make_async_copy(sbuf.at[s], out_hbm.at[pl.ds(tb, CH5)],
                              sem_out[s]).wait()

    start_in(0, 0)
    for chv in range(nch):
        s = chv % 2
        if chv + 1 < nch:
            if chv >= 1:
                wait_out(chv - 1, 1 - s)
            start_in(chv + 1, 1 - s)
        wait_in(chv, s)
        wv1 = wb1[pl.ds(chv * CH5, 16)]
        wv2 = wb2[pl.ds(chv * CH5, 16)]
        iota16 = lax.iota(jnp.int32, 16)
        zf = jnp.zeros((16,), jnp.float32)

        def row(r, _):
            wa = jnp.sum(jnp.where(iota16 == r, wv1, zf))
            wb = jnp.sum(jnp.where(iota16 == r, wv2, zf))

            def col(j, _):
                for q in range(4):
                    cs = pl.ds(j * 64 + q * 16, 16)
                    sbuf.at[s][r, cs] = (sbuf.at[s][r, cs]
                                         + wa * g1.at[s][r, cs]
                                         + wb * g2.at[s][r, cs])
                return 0

            lax.fori_loop(0, H // 64, col, 0)
            return 0

        lax.fori_loop(0, CH5, row, 0)
        tb = tbase + chv * CH5
        pltpu.async_copy(sbuf.at[s], out_hbm.at[pl.ds(tb, CH5)], sem_out[s])
    wait_out(nch - 2, nch % 2)
    wait_out(nch - 1, (nch - 1) % 2)


def _combine(sh, ys, pos, wp):
    mesh = plsc.VectorSubcoreMesh(core_axis_name="c", subcore_axis_name="s")
    f32 = jnp.float32
    f = pl.kernel(
        _combine_body,
        out_type=jax.ShapeDtypeStruct((T, H), f32),
        mesh=mesh,
        scratch_types=[
            pltpu.VMEM((2, CH5, H), f32),
            pltpu.VMEM((2, CH5, H), f32),
            pltpu.VMEM((2, CH5, H), f32),
            pltpu.VMEM((TPT,), jnp.int32),
            pltpu.VMEM((TPT,), jnp.int32),
            pltpu.VMEM((TPT + 16,), f32),
            pltpu.VMEM((TPT + 16,), f32),
            pltpu.SemaphoreType.DMA,
            pltpu.SemaphoreType.DMA,
            pltpu.SemaphoreType.DMA,
            pltpu.SemaphoreType.DMA,
        ],
        compiler_params=pltpu.CompilerParams(needs_layout_passes=False),
    )
    return f(sh, ys, pos, wp)


def kernel(x, shared_up, shared_down, routed_up, routed_down, router_w):
    up_bf = routed_up.astype(jnp.bfloat16)
    down_bf = routed_down.astype(jnp.bfloat16)
    u_sh = shared_up.reshape(N_SHARED * E_DIM, H).astype(jnp.bfloat16)
    d_sh = jnp.concatenate([shared_down[i] for i in range(N_SHARED)],
                           axis=1).astype(jnp.bfloat16)

    sh, eidx, prob = _shared_router(x, u_sh, d_sh, router_w)
    ep = jnp.concatenate([eidx[:, 0], eidx[:, 1]])
    wp = jnp.concatenate([prob[:, 0], prob[:, 1]])

    pos, vb, ve, vr0, vr1, xg = _dispatch(ep, x)
    ys = _gmm(vb, ve, vr0, vr1, xg, up_bf, down_bf)
    out = _combine(sh, ys, pos, wp)
    return out
